# Initial kernel scaffold; baseline (speedup 1.0000x reference)
import jax
import jax.numpy as jnp
from jax.experimental import pallas as pl


def kernel(X, Nbrs, Nbrs_Z, radial_params):
    raise NotImplementedError("write your pallas kernel here")


# speedup vs baseline: 17.1992x; 17.1992x over previous
import functools

import jax
import jax.numpy as jnp
import numpy as np
from jax import lax
from jax.experimental import pallas as pl
from jax.experimental.pallas import tpu as pltpu
from jax.experimental.pallas import tpu_sc as plsc

B, N, M = 32, 512, 32
L = 16
LT = L * 4
NLANE = 16
GROUPS = N // NLANE

_C0 = np.float32(9.99999444e-01)
_C1 = np.float32(-4.99995582e-01)
_C2 = np.float32(4.16610328e-02)
_C3 = np.float32(-1.38627473e-03)
_C4 = np.float32(2.42531925e-05)
_C5 = np.float32(-2.21939499e-07)
_PI = np.float32(np.pi)


def _sc_body(x_hbm, nb_hbm, z_hbm, rp_hbm, out_hbm,
             xv, nbv, zv, rpv, accv, rbuf, ibuf, zbuf):
    c = lax.axis_index("c")
    s = lax.axis_index("s")
    wid = s * 2 + c

    pltpu.sync_copy(x_hbm.at[wid], xv)
    pltpu.sync_copy(nb_hbm.at[wid], nbv)
    pltpu.sync_copy(z_hbm.at[wid], zv)
    pltpu.sync_copy(rp_hbm, rpv)

    zero = jnp.zeros((NLANE,), jnp.float32)

    def zbody(i, carry):
        accv[pl.ds(i * NLANE, NLANE)] = zero
        return carry

    lax.fori_loop(0, N * LT // NLANE, zbody, 0)

    lane = jnp.arange(NLANE, dtype=jnp.int32)

    def gbody(g, carry):
        atom = g * NLANE + lane
        a3 = atom * 3
        cx = plsc.load_gather(xv, [a3])
        cy = plsc.load_gather(xv, [a3 + 1])
        cz = plsc.load_gather(xv, [a3 + 2])
        base_off = atom * M
        acc_base = atom * LT

        def m1(m, carry1):
            off = base_off + m
            nb = plsc.load_gather(nbv, [off])
            zz = plsc.load_gather(zv, [off])
            nb3 = nb * 3
            dx = plsc.load_gather(xv, [nb3]) - cx
            dy = plsc.load_gather(xv, [nb3 + 1]) - cy
            dz = plsc.load_gather(xv, [nb3 + 2]) - cz
            r2 = dx * dx + dy * dy + dz * dz
            ii = lax.bitcast_convert_type(r2, jnp.int32)
            ii = jnp.int32(0x5F3759DF) - jnp.right_shift(ii, 1)
            y = lax.bitcast_convert_type(ii, jnp.float32)
            hr2 = np.float32(0.5) * r2
            y = y * (np.float32(1.5) - hr2 * y * y)
            y = y * (np.float32(1.5) - hr2 * y * y)
            y = y * (np.float32(1.5) - hr2 * y * y)
            r = r2 * y
            sl = pl.ds(m * NLANE, NLANE)
            rbuf[sl] = r
            ibuf[sl] = jnp.maximum(acc_base + (zz - 1), 0)
            zbuf[sl] = zz
            return carry1

        lax.fori_loop(0, M, m1, 0)

        def lbody(l, carry2):
            l3 = l * 3
            rc = plsc.load_gather(rpv, [jnp.full((NLANE,), l3, jnp.int32)])
            rs = plsc.load_gather(rpv, [jnp.full((NLANE,), l3 + 1, jnp.int32)])
            ee = plsc.load_gather(rpv, [jnp.full((NLANE,), l3 + 2, jnp.int32)])
            pinv = _PI / rc
            ne = -ee
            lt4 = l * 4

            def m2(m, carry3):
                sl = pl.ds(m * NLANE, NLANE)
                r = rbuf[sl]
                ib = ibuf[sl]
                zz = zbuf[sl]
                dd = r - rs
                kk = jnp.exp(ne * dd * dd)
                u = jnp.minimum(r * pinv, _PI)
                t = u * u
                cpoly = _C0 + t * (_C1 + t * (_C2 + t * (_C3 + t * (_C4 + t * _C5))))
                fc = jnp.where(r <= rc,
                               np.float32(0.5) * cpoly + np.float32(0.5),
                               np.float32(0.0))
                val = kk * fc
                mask = (zz >= 1) & (zz <= 4)
                plsc.addupdate_scatter(accv, [ib + lt4], val, mask=mask)
                return carry3

            lax.fori_loop(0, M, m2, 0)
            return carry2

        lax.fori_loop(0, L, lbody, 0)
        return carry

    lax.fori_loop(0, GROUPS, gbody, 0)

    pltpu.sync_copy(accv, out_hbm.at[wid])


_sc_main = functools.partial(
    pl.kernel,
    out_type=jax.ShapeDtypeStruct((B, N * LT), jnp.float32),
    mesh=plsc.VectorSubcoreMesh(core_axis_name="c", subcore_axis_name="s"),
    compiler_params=pltpu.CompilerParams(needs_layout_passes=False),
    scratch_types=[
        pltpu.VMEM((N * 3,), jnp.float32),
        pltpu.VMEM((N * M,), jnp.int32),
        pltpu.VMEM((N * M,), jnp.int32),
        pltpu.VMEM((L * 3,), jnp.float32),
        pltpu.VMEM((N * LT,), jnp.float32),
        pltpu.VMEM((M * NLANE,), jnp.float32),
        pltpu.VMEM((M * NLANE,), jnp.int32),
        pltpu.VMEM((M * NLANE,), jnp.int32),
    ],
)(_sc_body)


def _bn_body(x_ref, o_ref):
    x = x_ref[...]
    m = jnp.mean(x, axis=0, keepdims=True)
    d = x - m
    v = jnp.mean(d * d, axis=0, keepdims=True)
    o_ref[...] = d * lax.rsqrt(v + np.float32(0.001))


_BN_CHUNK = 2048


def _bn(layer):
    return pl.pallas_call(
        _bn_body,
        grid=(N * LT // _BN_CHUNK,),
        in_specs=[pl.BlockSpec((B, _BN_CHUNK), lambda i: (0, i))],
        out_specs=pl.BlockSpec((B, _BN_CHUNK), lambda i: (0, i)),
        out_shape=jax.ShapeDtypeStruct((B, N * LT), jnp.float32),
    )(layer)


def kernel(X, Nbrs, Nbrs_Z, radial_params):
    xf = X.reshape(B, N * 3)
    nb = Nbrs.reshape(B, N * M).astype(jnp.int32)
    zf = Nbrs_Z.reshape(B, N * M).astype(jnp.int32)
    rp = radial_params.reshape(L * 3)
    layer = _sc_main(xf, nb, zf, rp)
    out = _bn(layer)
    return out.reshape(B, N, LT)


# speedup vs baseline: 18.4981x; 1.0755x over previous
import functools

import jax
import jax.numpy as jnp
import numpy as np
from jax import lax
from jax.experimental import pallas as pl
from jax.experimental.pallas import tpu as pltpu
from jax.experimental.pallas import tpu_sc as plsc

B, N, M = 32, 512, 32
L = 16
LT = L * 4
NLANE = 16
GROUPS = N // NLANE

_C0 = np.float32(0.5 * 9.99999444e-01 + 0.5)
_C1 = np.float32(0.5 * -4.99995582e-01)
_C2 = np.float32(0.5 * 4.16610328e-02)
_C3 = np.float32(0.5 * -1.38627473e-03)
_C4 = np.float32(0.5 * 2.42531925e-05)
_C5 = np.float32(0.5 * -2.21939499e-07)
_PI = np.float32(np.pi)
_RBIG = np.float32(1e18)


def _sc_body(x_hbm, nb_hbm, z_hbm, rp_hbm, out_hbm,
             xv, nbv, zv, rpv, accv, rbuf, ibuf):
    c = lax.axis_index("c")
    s = lax.axis_index("s")
    wid = s * 2 + c

    pltpu.sync_copy(x_hbm.at[wid], xv)
    pltpu.sync_copy(nb_hbm.at[wid], nbv)
    pltpu.sync_copy(z_hbm.at[wid], zv)
    pltpu.sync_copy(rp_hbm, rpv)

    zero = jnp.zeros((NLANE,), jnp.float32)

    def zbody(i, carry):
        for k in range(8):
            accv[pl.ds((i * 8 + k) * NLANE, NLANE)] = zero
        return carry

    lax.fori_loop(0, N * LT // NLANE // 8, zbody, 0)

    lane = jnp.arange(NLANE, dtype=jnp.int32)

    def gbody(g, carry):
        atom = g * NLANE + lane
        a3 = atom * 3
        cx = plsc.load_gather(xv, [a3])
        cy = plsc.load_gather(xv, [a3 + 1])
        cz = plsc.load_gather(xv, [a3 + 2])
        base_off = atom * M
        acc_base = atom * LT

        def m1(mm, carry1):
            for k in range(2):
                m = mm * 2 + k
                off = base_off + m
                nb = plsc.load_gather(nbv, [off])
                zz = plsc.load_gather(zv, [off])
                nb3 = nb * 3
                dx = plsc.load_gather(xv, [nb3]) - cx
                dy = plsc.load_gather(xv, [nb3 + 1]) - cy
                dz = plsc.load_gather(xv, [nb3 + 2]) - cz
                r2 = dx * dx + dy * dy + dz * dz
                ii = lax.bitcast_convert_type(r2, jnp.int32)
                ii = jnp.int32(0x5F3759DF) - jnp.right_shift(ii, 1)
                y = lax.bitcast_convert_type(ii, jnp.float32)
                hr2 = np.float32(0.5) * r2
                y = y * (np.float32(1.5) - hr2 * y * y)
                y = y * (np.float32(1.5) - hr2 * y * y)
                y = y * (np.float32(1.5) - hr2 * y * y)
                r = r2 * y
                valid = (zz >= 1) & (zz <= 4)
                sl = pl.ds(m * NLANE, NLANE)
                rbuf[sl] = jnp.where(valid, r, _RBIG)
                ibuf[sl] = jnp.maximum(acc_base + (zz - 1), 0)
            return carry1

        lax.fori_loop(0, M // 2, m1, 0)

        def lbody(l, carry2):
            l3 = l * 3
            rc = plsc.load_gather(rpv, [jnp.full((NLANE,), l3, jnp.int32)])
            rs = plsc.load_gather(rpv, [jnp.full((NLANE,), l3 + 1, jnp.int32)])
            ee = plsc.load_gather(rpv, [jnp.full((NLANE,), l3 + 2, jnp.int32)])
            pinv = _PI / rc
            ne = -ee
            lt4 = l * 4

            def m2(mm, carry3):
                for k in range(4):
                    m = mm * 4 + k
                    sl = pl.ds(m * NLANE, NLANE)
                    r = rbuf[sl]
                    ib = ibuf[sl]
                    dd = r - rs
                    kk = jnp.exp(ne * dd * dd)
                    u = jnp.minimum(r * pinv, _PI)
                    t = u * u
                    fc = _C0 + t * (_C1 + t * (_C2 + t * (_C3 + t * (_C4 + t * _C5))))
                    fc = jnp.where(r <= rc, fc, np.float32(0.0))
                    val = kk * fc
                    plsc.addupdate_scatter(accv, [ib + lt4], val)
                return carry3

            lax.fori_loop(0, M // 4, m2, 0)
            return carry2

        lax.fori_loop(0, L, lbody, 0)
        return carry

    lax.fori_loop(0, GROUPS, gbody, 0)

    pltpu.sync_copy(accv, out_hbm.at[wid])


_sc_main = functools.partial(
    pl.kernel,
    out_type=jax.ShapeDtypeStruct((B, N * LT), jnp.float32),
    mesh=plsc.VectorSubcoreMesh(core_axis_name="c", subcore_axis_name="s"),
    compiler_params=pltpu.CompilerParams(needs_layout_passes=False),
    scratch_types=[
        pltpu.VMEM((N * 3,), jnp.float32),
        pltpu.VMEM((N * M,), jnp.int32),
        pltpu.VMEM((N * M,), jnp.int32),
        pltpu.VMEM((L * 3,), jnp.float32),
        pltpu.VMEM((N * LT,), jnp.float32),
        pltpu.VMEM((M * NLANE,), jnp.float32),
        pltpu.VMEM((M * NLANE,), jnp.int32),
    ],
)(_sc_body)


def _bn_body(x_ref, o_ref):
    x = x_ref[...]
    m = jnp.mean(x, axis=0, keepdims=True)
    d = x - m
    v = jnp.mean(d * d, axis=0, keepdims=True)
    o_ref[...] = d * lax.rsqrt(v + np.float32(0.001))


_BN_CHUNK = 2048


def _bn(layer):
    return pl.pallas_call(
        _bn_body,
        grid=(N * LT // _BN_CHUNK,),
        in_specs=[pl.BlockSpec((B, _BN_CHUNK), lambda i: (0, i))],
        out_specs=pl.BlockSpec((B, _BN_CHUNK), lambda i: (0, i)),
        out_shape=jax.ShapeDtypeStruct((B, N * LT), jnp.float32),
    )(layer)


def kernel(X, Nbrs, Nbrs_Z, radial_params):
    xf = X.reshape(B, N * 3)
    nb = Nbrs.reshape(B, N * M).astype(jnp.int32)
    zf = Nbrs_Z.reshape(B, N * M).astype(jnp.int32)
    rp = radial_params.reshape(L * 3)
    layer = _sc_main(xf, nb, zf, rp)
    out = _bn(layer)
    return out.reshape(B, N, LT)


# speedup vs baseline: 47.0889x; 2.5456x over previous
import functools

import jax
import jax.numpy as jnp
import numpy as np
from jax import lax
from jax.experimental import pallas as pl
from jax.experimental.pallas import tpu as pltpu
from jax.experimental.pallas import tpu_sc as plsc

B, N, M = 32, 512, 32
L = 16
LT = L * 4
NLANE = 16
GROUPS = N // NLANE

_C0 = np.float32(0.5 * 9.99999444e-01 + 0.5)
_C1 = np.float32(0.5 * -4.99995582e-01)
_C2 = np.float32(0.5 * 4.16610328e-02)
_C3 = np.float32(0.5 * -1.38627473e-03)
_C4 = np.float32(0.5 * 2.42531925e-05)
_C5 = np.float32(0.5 * -2.21939499e-07)
_PI = np.float32(np.pi)
_RBIG = np.float32(1e18)


def _sc_body(x_hbm, nb_hbm, z_hbm, rp_hbm, out_hbm,
             xv, nbv, zv, rpv, accv, rbuf, ibuf):
    c = lax.axis_index("c")
    s = lax.axis_index("s")
    wid = s * 2 + c

    pltpu.sync_copy(x_hbm.at[wid], xv)
    pltpu.sync_copy(nb_hbm.at[wid], nbv)
    pltpu.sync_copy(z_hbm.at[wid], zv)
    pltpu.sync_copy(rp_hbm, rpv)

    zero = jnp.zeros((NLANE,), jnp.float32)

    @plsc.parallel_loop(0, N * LT // NLANE, unroll=8)
    def zbody(i):
        accv[pl.ds(i * NLANE, NLANE)] = zero

    lane = jnp.arange(NLANE, dtype=jnp.int32)

    def gbody(g, carry):
        atom = g * NLANE + lane
        a3 = atom * 3
        cx = plsc.load_gather(xv, [a3])
        cy = plsc.load_gather(xv, [a3 + 1])
        cz = plsc.load_gather(xv, [a3 + 2])
        base_off = atom * M
        acc_base = atom * LT

        @plsc.parallel_loop(0, M, unroll=2)
        def m1(m):
            off = base_off + m
            nb = plsc.load_gather(nbv, [off])
            zz = plsc.load_gather(zv, [off])
            nb3 = nb * 3
            dx = plsc.load_gather(xv, [nb3]) - cx
            dy = plsc.load_gather(xv, [nb3 + 1]) - cy
            dz = plsc.load_gather(xv, [nb3 + 2]) - cz
            r2 = dx * dx + dy * dy + dz * dz
            ii = lax.bitcast_convert_type(r2, jnp.int32)
            ii = jnp.int32(0x5F3759DF) - jnp.right_shift(ii, 1)
            y = lax.bitcast_convert_type(ii, jnp.float32)
            hr2 = np.float32(0.5) * r2
            y = y * (np.float32(1.5) - hr2 * y * y)
            y = y * (np.float32(1.5) - hr2 * y * y)
            y = y * (np.float32(1.5) - hr2 * y * y)
            r = r2 * y
            valid = (zz >= 1) & (zz <= 4)
            sl = pl.ds(m * NLANE, NLANE)
            rbuf[sl] = jnp.where(valid, r, _RBIG)
            ibuf[sl] = jnp.maximum(acc_base + (zz - 1), 0)

        def lbody(l, carry2):
            l3 = l * 3
            rc = plsc.load_gather(rpv, [jnp.full((NLANE,), l3, jnp.int32)])
            rs = plsc.load_gather(rpv, [jnp.full((NLANE,), l3 + 1, jnp.int32)])
            ee = plsc.load_gather(rpv, [jnp.full((NLANE,), l3 + 2, jnp.int32)])
            pinv = _PI / rc
            ne = -ee
            lt4 = l * 4

            @plsc.parallel_loop(0, M, unroll=4)
            def m2(m):
                sl = pl.ds(m * NLANE, NLANE)
                r = rbuf[sl]
                ib = ibuf[sl]
                dd = r - rs
                kk = jnp.exp(ne * dd * dd)
                u = jnp.minimum(r * pinv, _PI)
                t = u * u
                fc = _C0 + t * (_C1 + t * (_C2 + t * (_C3 + t * (_C4 + t * _C5))))
                fc = jnp.where(r <= rc, fc, np.float32(0.0))
                val = kk * fc
                plsc.addupdate_scatter(accv, [ib + lt4], val)

            return carry2

        lax.fori_loop(0, L, lbody, 0)
        return carry

    lax.fori_loop(0, GROUPS, gbody, 0)

    pltpu.sync_copy(accv, out_hbm.at[wid])


_sc_main = functools.partial(
    pl.kernel,
    out_type=jax.ShapeDtypeStruct((B, N * LT), jnp.float32),
    mesh=plsc.VectorSubcoreMesh(core_axis_name="c", subcore_axis_name="s"),
    compiler_params=pltpu.CompilerParams(needs_layout_passes=False),
    scratch_types=[
        pltpu.VMEM((N * 3,), jnp.float32),
        pltpu.VMEM((N * M,), jnp.int32),
        pltpu.VMEM((N * M,), jnp.int32),
        pltpu.VMEM((L * 3,), jnp.float32),
        pltpu.VMEM((N * LT,), jnp.float32),
        pltpu.VMEM((M * NLANE,), jnp.float32),
        pltpu.VMEM((M * NLANE,), jnp.int32),
    ],
)(_sc_body)


def _bn_body(x_ref, o_ref):
    x = x_ref[...]
    m = jnp.mean(x, axis=0, keepdims=True)
    d = x - m
    v = jnp.mean(d * d, axis=0, keepdims=True)
    o_ref[...] = d * lax.rsqrt(v + np.float32(0.001))


_BN_CHUNK = 2048


def _bn(layer):
    return pl.pallas_call(
        _bn_body,
        grid=(N * LT // _BN_CHUNK,),
        in_specs=[pl.BlockSpec((B, _BN_CHUNK), lambda i: (0, i))],
        out_specs=pl.BlockSpec((B, _BN_CHUNK), lambda i: (0, i)),
        out_shape=jax.ShapeDtypeStruct((B, N * LT), jnp.float32),
    )(layer)


def kernel(X, Nbrs, Nbrs_Z, radial_params):
    xf = X.reshape(B, N * 3)
    nb = Nbrs.reshape(B, N * M).astype(jnp.int32)
    zf = Nbrs_Z.reshape(B, N * M).astype(jnp.int32)
    rp = radial_params.reshape(L * 3)
    layer = _sc_main(xf, nb, zf, rp)
    out = _bn(layer)
    return out.reshape(B, N, LT)


# speedup vs baseline: 49.9335x; 1.0604x over previous
import functools

import jax
import jax.numpy as jnp
import numpy as np
from jax import lax
from jax.experimental import pallas as pl
from jax.experimental.pallas import tpu as pltpu
from jax.experimental.pallas import tpu_sc as plsc

B, N, M = 32, 512, 32
L = 16
LT = L * 4
NLANE = 16
GROUPS = N // NLANE

_C0 = np.float32(0.9999855460225976)
_C1 = np.float32(-0.24991878892384348)
_C2 = np.float32(0.02076114541288175)
_C3 = np.float32(-0.0006720519689284828)
_C4 = np.float32(9.532515384707557e-06)
_PI = np.float32(np.pi)
_LOG2E = np.float32(1.4426950408889634)
_RBIG = np.float32(1e18)


def _sc_body(x_hbm, nb_hbm, z_hbm, rp_hbm, out_hbm,
             xv, nbv, zv, rpv, accv, rbuf, ibuf, pbuf):
    c = lax.axis_index("c")
    s = lax.axis_index("s")
    wid = s * 2 + c

    pltpu.sync_copy(x_hbm.at[wid], xv)
    pltpu.sync_copy(nb_hbm.at[wid], nbv)
    pltpu.sync_copy(z_hbm.at[wid], zv)
    pltpu.sync_copy(rp_hbm, rpv)

    zero = jnp.zeros((NLANE,), jnp.float32)

    @plsc.parallel_loop(0, N * LT // NLANE, unroll=8)
    def zbody(i):
        accv[pl.ds(i * NLANE, NLANE)] = zero

    lane = jnp.arange(NLANE, dtype=jnp.int32)

    rc_all = plsc.load_gather(rpv, [lane * 3])
    rs_all = plsc.load_gather(rpv, [lane * 3 + 1])
    e_all = plsc.load_gather(rpv, [lane * 3 + 2])
    pbuf[pl.ds(0, NLANE)] = rc_all
    pbuf[pl.ds(NLANE, NLANE)] = rs_all
    pbuf[pl.ds(2 * NLANE, NLANE)] = -e_all
    pbuf[pl.ds(3 * NLANE, NLANE)] = _PI / rc_all

    def gbody(g, carry):
        atom = g * NLANE + lane
        a3 = atom * 3
        cx = plsc.load_gather(xv, [a3])
        cy = plsc.load_gather(xv, [a3 + 1])
        cz = plsc.load_gather(xv, [a3 + 2])
        base_off = atom * M
        acc_base = atom * LT

        @plsc.parallel_loop(0, M, unroll=4)
        def m1(m):
            off = base_off + m
            nb = plsc.load_gather(nbv, [off])
            zz = plsc.load_gather(zv, [off])
            nb3 = nb * 3
            dx = plsc.load_gather(xv, [nb3]) - cx
            dy = plsc.load_gather(xv, [nb3 + 1]) - cy
            dz = plsc.load_gather(xv, [nb3 + 2]) - cz
            r2 = dx * dx + dy * dy + dz * dz
            ii = lax.bitcast_convert_type(r2, jnp.int32)
            ii = jnp.int32(0x5F3759DF) - jnp.right_shift(ii, 1)
            y = lax.bitcast_convert_type(ii, jnp.float32)
            hr2 = np.float32(0.5) * r2
            y = y * (np.float32(1.5) - hr2 * y * y)
            y = y * (np.float32(1.5) - hr2 * y * y)
            y = y * (np.float32(1.5) - hr2 * y * y)
            r = r2 * y
            valid = (zz >= 1) & (zz <= 4)
            sl = pl.ds(m * NLANE, NLANE)
            rbuf[sl] = jnp.where(valid, r, _RBIG)
            ibuf[sl] = jnp.maximum(acc_base + (zz - 1), 0)

        def lbody(l, carry2):
            rc = plsc.load_gather(pbuf, [jnp.full((NLANE,), l, jnp.int32)])
            rs = plsc.load_gather(pbuf, [jnp.full((NLANE,), l + NLANE, jnp.int32)])
            ne = plsc.load_gather(pbuf, [jnp.full((NLANE,), l + 2 * NLANE, jnp.int32)])
            pinv = plsc.load_gather(pbuf, [jnp.full((NLANE,), l + 3 * NLANE, jnp.int32)])
            lt4 = l * 4

            @plsc.parallel_loop(0, M, unroll=4)
            def m2(m):
                sl = pl.ds(m * NLANE, NLANE)
                r = rbuf[sl]
                ib = ibuf[sl]
                dd = r - rs
                kk = jnp.exp(ne * dd * dd)
                u = r * pinv
                t = u * u
                fc = _C0 + t * (_C1 + t * (_C2 + t * (_C3 + t * _C4)))
                fc = jnp.where(r <= rc, fc, np.float32(0.0))
                val = kk * fc
                plsc.addupdate_scatter(accv, [ib + lt4], val)

            return carry2

        lax.fori_loop(0, L, lbody, 0)
        return carry

    lax.fori_loop(0, GROUPS, gbody, 0)

    pltpu.sync_copy(accv, out_hbm.at[wid])


_sc_main = functools.partial(
    pl.kernel,
    out_type=jax.ShapeDtypeStruct((B, N * LT), jnp.float32),
    mesh=plsc.VectorSubcoreMesh(core_axis_name="c", subcore_axis_name="s"),
    compiler_params=pltpu.CompilerParams(needs_layout_passes=False),
    scratch_types=[
        pltpu.VMEM((N * 3,), jnp.float32),
        pltpu.VMEM((N * M,), jnp.int32),
        pltpu.VMEM((N * M,), jnp.int32),
        pltpu.VMEM((L * 3,), jnp.float32),
        pltpu.VMEM((N * LT,), jnp.float32),
        pltpu.VMEM((M * NLANE,), jnp.float32),
        pltpu.VMEM((M * NLANE,), jnp.int32),
        pltpu.VMEM((4 * NLANE,), jnp.float32),
    ],
)(_sc_body)


def _bn_body(x_ref, o_ref):
    x = x_ref[...]
    m = jnp.mean(x, axis=0, keepdims=True)
    d = x - m
    v = jnp.mean(d * d, axis=0, keepdims=True)
    o_ref[...] = d * lax.rsqrt(v + np.float32(0.001))


_BN_CHUNK = 2048


def _bn(layer):
    return pl.pallas_call(
        _bn_body,
        grid=(N * LT // _BN_CHUNK,),
        in_specs=[pl.BlockSpec((B, _BN_CHUNK), lambda i: (0, i))],
        out_specs=pl.BlockSpec((B, _BN_CHUNK), lambda i: (0, i)),
        out_shape=jax.ShapeDtypeStruct((B, N * LT), jnp.float32),
    )(layer)


def kernel(X, Nbrs, Nbrs_Z, radial_params):
    xf = X.reshape(B, N * 3)
    nb = Nbrs.reshape(B, N * M).astype(jnp.int32)
    zf = Nbrs_Z.reshape(B, N * M).astype(jnp.int32)
    rp = radial_params.reshape(L * 3)
    layer = _sc_main(xf, nb, zf, rp)
    out = _bn(layer)
    return out.reshape(B, N, LT)
